# baseline (speedup 1.0000x reference)
import jax
import jax.numpy as jnp
from jax.experimental import pallas as pl
from jax.experimental.pallas import tpu as pltpu

_N = 20000
_NPAD = 20480
_ROWS = 160
_PRE = 1000
_PREPAD = 1024
_POST = 100
_NMS_T = 0.6
_SCORE_T = 0.05
_NEG = -1e9


_NB = 16


def _nms_body(s_ref, x1_ref, y1_ref, x2_ref, y2_ref, out_ref,
              sw, ss, bsx1, bsy1, bsx2, bsy2,
              bcx1, bcy1, bcx2, bcy2, mm, mdiag, keep, cnt,
              *grids):
    f32 = jnp.float32

    sw[...] = jnp.where(s_ref[...] > _SCORE_T, s_ref[...], _NEG)
    zcol = jnp.zeros((_PREPAD, 1), f32)
    bcx1[...] = zcol
    bcy1[...] = zcol
    bcx2[...] = zcol
    bcy2[...] = zcol
    zrow = jnp.zeros((1, _PREPAD), f32)
    bsx1[...] = zrow
    bsy1[...] = zrow
    bsx2[...] = zrow
    bsy2[...] = zrow
    ss[...] = jnp.full((1, _PREPAD), _NEG, f32)
    keep[...] = jnp.ones((1, _PREPAD), f32)
    cnt[0] = 0

    iota_r = jax.lax.broadcasted_iota(jnp.int32, (1, _ROWS), 1)
    iota_c = jax.lax.broadcasted_iota(jnp.int32, (1, 128), 1)
    iota_s8 = jax.lax.broadcasted_iota(jnp.int32, (8, 1), 0)
    big = jnp.int32(1 << 20)

    NB = _NB
    tgt = float(_PRE) / NB

    def bis_round(_, carry):
        los, his = carry
        svals = sw[...]
        nlo = []
        nhi = []
        for k in range(NB - 1):
            mid = (los[k] + his[k]) * 0.5
            cntk = jnp.sum(jnp.where(svals > mid, 1.0, 0.0))
            up = cntk > (tgt * (k + 1))
            nlo.append(jnp.where(up, mid, los[k]))
            nhi.append(jnp.where(up, his[k], mid))
        return (tuple(nlo), tuple(nhi))

    los0 = tuple(jnp.float32(0.0) for _ in range(NB - 1))
    his0 = tuple(jnp.float32(1.0) for _ in range(NB - 1))
    los, _ = jax.lax.fori_loop(0, 36, bis_round, (los0, his0))

    ts = []
    prev = jnp.float32(2.0)
    for k in range(NB - 1):
        tk = jnp.minimum(los[k], prev)
        ts.append(tk)
        prev = tk

    svals = sw[...]
    cs = [jnp.sum(jnp.where(svals > t, 1.0, 0.0)).astype(jnp.int32)
          for t in ts]
    starts = [jnp.int32(0)] + cs
    ns = []
    for b in range(NB):
        if b < NB - 1:
            nb_ = cs[b] - starts[b]
        else:
            nb_ = jnp.maximum(_PRE - cs[NB - 2], 0)
        ns.append(jnp.minimum(nb_, _PRE))
    T = ns[0]
    for b in range(1, NB):
        T = jnp.maximum(T, ns[b])

    for b in range(NB):
        cond = None
        if b > 0:
            cond = svals <= ts[b - 1]
        if b < NB - 1:
            c2 = svals > ts[b]
            cond = c2 if cond is None else (cond & c2)
        grids[b][...] = jnp.where(cond, svals, -jnp.inf)

    rm0s = tuple(
        jnp.reshape(jnp.max(grids[b][...], axis=1), (1, _ROWS))
        for b in range(NB))

    def extractT(i, rms):
        new_rms = []
        for b in range(NB):
            gk = grids[b]
            rmv = rms[b]
            gm = jnp.max(rmv, axis=1, keepdims=True)
            row_vec = jnp.min(jnp.where(rmv == gm, iota_r, big),
                              axis=1, keepdims=True)
            row = row_vec[0, 0]
            rowv = gk[pl.ds(row, 1), :]
            col_vec = jnp.min(jnp.where(rowv == gm, iota_c, big),
                              axis=1, keepdims=True)
            cmask = iota_c == col_vec
            new_rowv = jnp.where(cmask, -jnp.inf, rowv)
            gk[pl.ds(row, 1), :] = new_rowv
            new_rms.append(
                jnp.where(iota_r == row_vec,
                          jnp.max(new_rowv, axis=1, keepdims=True), rmv))
            x1v = jnp.sum(jnp.where(cmask, x1_ref[pl.ds(row, 1), :], 0.0),
                          axis=1, keepdims=True)
            y1v = jnp.sum(jnp.where(cmask, y1_ref[pl.ds(row, 1), :], 0.0),
                          axis=1, keepdims=True)
            x2v = jnp.sum(jnp.where(cmask, x2_ref[pl.ds(row, 1), :], 0.0),
                          axis=1, keepdims=True)
            y2v = jnp.sum(jnp.where(cmask, y2_ref[pl.ds(row, 1), :], 0.0),
                          axis=1, keepdims=True)
            pos = starts[b] + i
            ok = (i < ns[b]) & (pos < _PRE)
            posw = jnp.where(ok, pos, jnp.int32(1016))
            blk = pl.multiple_of((posw // 128) * 128, 128)
            lmw = iota_c == (posw - blk)
            for ref, val in ((ss, gm), (bsx1, x1v), (bsy1, y1v),
                             (bsx2, x2v), (bsy2, y2v)):
                cur = ref[0:1, pl.ds(blk, 128)]
                ref[0:1, pl.ds(blk, 128)] = jnp.where(lmw, val, cur)
            r8 = pl.multiple_of((posw // 8) * 8, 8)
            smw = iota_s8 == (posw - r8)
            for ref, val in ((bcx1, x1v), (bcy1, y1v),
                             (bcx2, x2v), (bcy2, y2v)):
                cur = ref[pl.ds(r8, 8), :]
                ref[pl.ds(r8, 8), :] = jnp.where(smw, val, cur)
        return tuple(new_rms)

    jax.lax.fori_loop(0, T, extractT, rm0s)

    def iou_tile(t, _):
        r0 = t * 8
        xi1 = bcx1[pl.ds(r0, 8), :]
        yi1 = bcy1[pl.ds(r0, 8), :]
        xi2 = bcx2[pl.ds(r0, 8), :]
        yi2 = bcy2[pl.ds(r0, 8), :]
        ai = (xi2 - xi1) * (yi2 - yi1)
        xj1 = bsx1[0:1, :]
        yj1 = bsy1[0:1, :]
        xj2 = bsx2[0:1, :]
        yj2 = bsy2[0:1, :]
        aj = (xj2 - xj1) * (yj2 - yj1)
        w = jnp.maximum(jnp.minimum(xi2, xj2) - jnp.maximum(xi1, xj1), 0.0)
        h = jnp.maximum(jnp.minimum(yi2, yj2) - jnp.maximum(yi1, yj1), 0.0)
        inter = w * h
        iou = inter / (ai + aj - inter + 1e-9)
        mm[pl.ds(r0, 8), :] = (iou > _NMS_T).astype(f32)
        bd0 = pl.multiple_of((t // 16) * 128, 128)
        xj1d = bsx1[0:1, pl.ds(bd0, 128)]
        yj1d = bsy1[0:1, pl.ds(bd0, 128)]
        xj2d = bsx2[0:1, pl.ds(bd0, 128)]
        yj2d = bsy2[0:1, pl.ds(bd0, 128)]
        ajd = (xj2d - xj1d) * (yj2d - yj1d)
        wd = jnp.maximum(jnp.minimum(xi2, xj2d) - jnp.maximum(xi1, xj1d), 0.0)
        hd = jnp.maximum(jnp.minimum(yi2, yj2d) - jnp.maximum(yi1, yj1d), 0.0)
        interd = wd * hd
        ioud = interd / (ai + ajd - interd + 1e-9)
        mdiag[pl.ds(r0, 8), :] = (ioud > _NMS_T).astype(f32)
        return 0

    jax.lax.fori_loop(0, _PREPAD // 8, iou_tile, 0)

    iota_j = jax.lax.broadcasted_iota(jnp.int32, (1, _PREPAD), 1)

    def nms_block(b, _):
        b0 = pl.multiple_of(b * 128, 128)

        iota_s128 = jax.lax.broadcasted_iota(jnp.int32, (128, 1), 0)

        def step(k, kb):
            i = b0 + k
            ek = (iota_s128 == k).astype(f32)
            ki = jax.lax.dot_general(kb, ek, (((1,), (0,)), ((), ())),
                                     preferred_element_type=f32)
            mrow = mdiag[pl.ds(i, 1), :]
            gtk = (iota_c > k).astype(f32)
            kb = kb * (1.0 - mrow * gtk * ki)
            return kb

        kb = jax.lax.fori_loop(0, 128, step, keep[0:1, pl.ds(b0, 128)])
        keep[0:1, pl.ds(b0, 128)] = kb
        mblock = mm[pl.ds(b0, 128), :]
        supp = jax.lax.dot_general(kb, mblock, (((1,), (0,)), ((), ())),
                                   preferred_element_type=f32)
        kv = keep[0:1, :]
        keep[0:1, :] = jnp.where((iota_j >= b0 + 128) & (supp > 0.0),
                                 0.0, kv)
        return 0

    jax.lax.fori_loop(0, _PREPAD // 128, nms_block, 0)

    kv = keep[0:1, :]
    ssv = ss[0:1, :]
    xr1 = bsx1[0:1, :]
    yr1 = bsy1[0:1, :]
    xr2 = bsx2[0:1, :]
    yr2 = bsy2[0:1, :]
    valid = iota_j < _PRE
    cond1 = (kv > 0.5) & (ssv > -5e8) & valid
    elig1 = jnp.where(cond1, 1.0, 0.0)
    elig2 = jnp.where(valid & (~cond1), 1.0, 0.0)

    def emit(p, carry):
        e1, e2 = carry
        keyv = jnp.where(e1 > 0.0, iota_j,
                         jnp.where(e2 > 0.0, iota_j + 2048, big))
        t = jnp.min(keyv, axis=1, keepdims=True)
        from1 = t < 2048
        lane = jnp.where(from1, t, t - 2048)
        lmask = iota_j == lane
        si = jnp.sum(jnp.where(lmask, ssv, 0.0), axis=1, keepdims=True)
        score = jnp.where(from1, si, jnp.float32(_NEG))
        x1v = jnp.sum(jnp.where(lmask, xr1, 0.0), axis=1, keepdims=True)
        y1v = jnp.sum(jnp.where(lmask, yr1, 0.0), axis=1, keepdims=True)
        x2v = jnp.sum(jnp.where(lmask, xr2, 0.0), axis=1, keepdims=True)
        y2v = jnp.sum(jnp.where(lmask, yr2, 0.0), axis=1, keepdims=True)
        rowout = jnp.where(
            iota_c == 0, x1v,
            jnp.where(iota_c == 1, y1v,
                      jnp.where(iota_c == 2, x2v,
                                jnp.where(iota_c == 3, y2v, score))))
        out_ref[pl.ds(p, 1), :] = rowout
        notl = jnp.where(lmask, 0.0, 1.0)
        return (e1 * notl, e2 * notl)

    jax.lax.fori_loop(0, _POST, emit, (elig1, elig2))


def _run(boxes, scores, interpret=False):
    f32 = jnp.float32
    pad = _NPAD - _N
    s2d = jnp.pad(scores, (0, pad)).reshape(_ROWS, 128)
    planes = [jnp.pad(boxes[:, k], (0, pad)).reshape(_ROWS, 128)
              for k in range(4)]
    out = pl.pallas_call(
        _nms_body,
        out_shape=jax.ShapeDtypeStruct((_POST, 128), f32),
        scratch_shapes=[
            pltpu.VMEM((_ROWS, 128), f32),
            pltpu.VMEM((1, _PREPAD), f32),
            pltpu.VMEM((1, _PREPAD), f32),
            pltpu.VMEM((1, _PREPAD), f32),
            pltpu.VMEM((1, _PREPAD), f32),
            pltpu.VMEM((1, _PREPAD), f32),
            pltpu.VMEM((_PREPAD, 1), f32),
            pltpu.VMEM((_PREPAD, 1), f32),
            pltpu.VMEM((_PREPAD, 1), f32),
            pltpu.VMEM((_PREPAD, 1), f32),
            pltpu.VMEM((_PREPAD, _PREPAD), f32),
            pltpu.VMEM((_PREPAD, 128), f32),
            pltpu.VMEM((1, _PREPAD), f32),
            pltpu.SMEM((1,), jnp.int32),
        ] + [pltpu.VMEM((_ROWS, 128), f32) for _ in range(_NB)],
        interpret=interpret,
    )(s2d, *planes)
    return out[:, :5]


@jax.jit
def _run_compiled(boxes, scores):
    return _run(boxes, scores)


def kernel(boxes, scores):
    return _run_compiled(boxes, scores)


# speedup vs baseline: 3.3389x; 3.3389x over previous
import jax
import jax.numpy as jnp
from jax.experimental import pallas as pl
from jax.experimental.pallas import tpu as pltpu

_N = 20000
_NPAD = 20480
_ROWS = 160
_PRE = 1000
_PREPAD = 1024
_POST = 100
_NMS_T = 0.6
_SCORE_T = 0.05
_NEG = -1e9


_NB = 32


def _nms_body(s_ref, x1_ref, y1_ref, x2_ref, y2_ref, out_ref,
              sw, ss, bsx1, bsy1, bsx2, bsy2,
              bcx1, bcy1, bcx2, bcy2, mm, mdiag, keep, cnt,
              *grids):
    f32 = jnp.float32

    sw[...] = jnp.where(s_ref[...] > _SCORE_T, s_ref[...], _NEG)
    zcol = jnp.zeros((_PREPAD, 1), f32)
    bcx1[...] = zcol
    bcy1[...] = zcol
    bcx2[...] = zcol
    bcy2[...] = zcol
    zrow = jnp.zeros((1, _PREPAD), f32)
    bsx1[...] = zrow
    bsy1[...] = zrow
    bsx2[...] = zrow
    bsy2[...] = zrow
    ss[...] = jnp.full((1, _PREPAD), _NEG, f32)
    keep[...] = jnp.ones((1, _PREPAD), f32)
    cnt[0] = 0

    iota_r = jax.lax.broadcasted_iota(jnp.int32, (1, _ROWS), 1)
    iota_c = jax.lax.broadcasted_iota(jnp.int32, (1, 128), 1)
    iota_s8 = jax.lax.broadcasted_iota(jnp.int32, (8, 1), 0)
    big = jnp.int32(1 << 20)

    NB = _NB
    tgt = float(_PRE) / NB

    def bis_round(_, carry):
        los, his = carry
        svals = sw[...]
        nlo = []
        nhi = []
        for k in range(NB - 1):
            mid = (los[k] + his[k]) * 0.5
            cntk = jnp.sum(jnp.where(svals > mid, 1.0, 0.0))
            up = cntk > (tgt * (k + 1))
            nlo.append(jnp.where(up, mid, los[k]))
            nhi.append(jnp.where(up, his[k], mid))
        return (tuple(nlo), tuple(nhi))

    los0 = tuple(jnp.float32(0.0) for _ in range(NB - 1))
    his0 = tuple(jnp.float32(1.0) for _ in range(NB - 1))
    los, _ = jax.lax.fori_loop(0, 28, bis_round, (los0, his0))

    ts = []
    prev = jnp.float32(2.0)
    for k in range(NB - 1):
        tk = jnp.minimum(los[k], prev)
        ts.append(tk)
        prev = tk

    svals = sw[...]
    cs = [jnp.sum(jnp.where(svals > t, 1.0, 0.0)).astype(jnp.int32)
          for t in ts]
    starts = [jnp.int32(0)] + cs
    ns = []
    for b in range(NB):
        if b < NB - 1:
            nb_ = cs[b] - starts[b]
        else:
            nb_ = jnp.maximum(_PRE - cs[NB - 2], 0)
        ns.append(jnp.minimum(nb_, _PRE))
    T = ns[0]
    for b in range(1, NB):
        T = jnp.maximum(T, ns[b])

    for b in range(NB):
        cond = None
        if b > 0:
            cond = svals <= ts[b - 1]
        if b < NB - 1:
            c2 = svals > ts[b]
            cond = c2 if cond is None else (cond & c2)
        grids[b][...] = jnp.where(cond, svals, -jnp.inf)

    rm0s = tuple(
        jnp.reshape(jnp.max(grids[b][...], axis=1), (1, _ROWS))
        for b in range(NB))

    def extractT(i, rms):
        new_rms = []
        for b in range(NB):
            gk = grids[b]
            rmv = rms[b]
            gm = jnp.max(rmv, axis=1, keepdims=True)
            row_vec = jnp.min(jnp.where(rmv == gm, iota_r, big),
                              axis=1, keepdims=True)
            row = row_vec[0, 0]
            rowv = gk[pl.ds(row, 1), :]
            col_vec = jnp.min(jnp.where(rowv == gm, iota_c, big),
                              axis=1, keepdims=True)
            cmask = iota_c == col_vec
            new_rowv = jnp.where(cmask, -jnp.inf, rowv)
            gk[pl.ds(row, 1), :] = new_rowv
            new_rms.append(
                jnp.where(iota_r == row_vec,
                          jnp.max(new_rowv, axis=1, keepdims=True), rmv))
            x1v = jnp.sum(jnp.where(cmask, x1_ref[pl.ds(row, 1), :], 0.0),
                          axis=1, keepdims=True)
            y1v = jnp.sum(jnp.where(cmask, y1_ref[pl.ds(row, 1), :], 0.0),
                          axis=1, keepdims=True)
            x2v = jnp.sum(jnp.where(cmask, x2_ref[pl.ds(row, 1), :], 0.0),
                          axis=1, keepdims=True)
            y2v = jnp.sum(jnp.where(cmask, y2_ref[pl.ds(row, 1), :], 0.0),
                          axis=1, keepdims=True)
            pos = starts[b] + i
            ok = (i < ns[b]) & (pos < _PRE)
            posw = jnp.where(ok, pos, jnp.int32(1016))
            blk = pl.multiple_of((posw // 128) * 128, 128)
            lmw = iota_c == (posw - blk)
            for ref, val in ((ss, gm), (bsx1, x1v), (bsy1, y1v),
                             (bsx2, x2v), (bsy2, y2v)):
                cur = ref[0:1, pl.ds(blk, 128)]
                ref[0:1, pl.ds(blk, 128)] = jnp.where(lmw, val, cur)
            r8 = pl.multiple_of((posw // 8) * 8, 8)
            smw = iota_s8 == (posw - r8)
            for ref, val in ((bcx1, x1v), (bcy1, y1v),
                             (bcx2, x2v), (bcy2, y2v)):
                cur = ref[pl.ds(r8, 8), :]
                ref[pl.ds(r8, 8), :] = jnp.where(smw, val, cur)
        return tuple(new_rms)

    jax.lax.fori_loop(0, T, extractT, rm0s)

    def iou_tile(t, _):
        r0 = t * 8
        xi1 = bcx1[pl.ds(r0, 8), :]
        yi1 = bcy1[pl.ds(r0, 8), :]
        xi2 = bcx2[pl.ds(r0, 8), :]
        yi2 = bcy2[pl.ds(r0, 8), :]
        ai = (xi2 - xi1) * (yi2 - yi1)
        xj1 = bsx1[0:1, :]
        yj1 = bsy1[0:1, :]
        xj2 = bsx2[0:1, :]
        yj2 = bsy2[0:1, :]
        aj = (xj2 - xj1) * (yj2 - yj1)
        w = jnp.maximum(jnp.minimum(xi2, xj2) - jnp.maximum(xi1, xj1), 0.0)
        h = jnp.maximum(jnp.minimum(yi2, yj2) - jnp.maximum(yi1, yj1), 0.0)
        inter = w * h
        iou = inter / (ai + aj - inter + 1e-9)
        mm[pl.ds(r0, 8), :] = (iou > _NMS_T).astype(f32)
        bd0 = pl.multiple_of((t // 16) * 128, 128)
        xj1d = bsx1[0:1, pl.ds(bd0, 128)]
        yj1d = bsy1[0:1, pl.ds(bd0, 128)]
        xj2d = bsx2[0:1, pl.ds(bd0, 128)]
        yj2d = bsy2[0:1, pl.ds(bd0, 128)]
        ajd = (xj2d - xj1d) * (yj2d - yj1d)
        wd = jnp.maximum(jnp.minimum(xi2, xj2d) - jnp.maximum(xi1, xj1d), 0.0)
        hd = jnp.maximum(jnp.minimum(yi2, yj2d) - jnp.maximum(yi1, yj1d), 0.0)
        interd = wd * hd
        ioud = interd / (ai + ajd - interd + 1e-9)
        mdiag[pl.ds(r0, 8), :] = (ioud > _NMS_T).astype(f32)
        return 0

    jax.lax.fori_loop(0, _PREPAD // 8, iou_tile, 0)

    iota_j = jax.lax.broadcasted_iota(jnp.int32, (1, _PREPAD), 1)

    def nms_block(b, _):
        b0 = pl.multiple_of(b * 128, 128)

        def step4(g, kb):
            k = g * 4
            i = b0 + k
            mrow = [mdiag[pl.ds(i + u, 1), :] for u in range(4)]

            def lane(vec, off):
                return jnp.sum(jnp.where(iota_c == off, vec, 0.0),
                               axis=1, keepdims=True)

            a = lane(kb, k)
            b_0 = lane(kb, k + 1)
            c_0 = lane(kb, k + 2)
            d_0 = lane(kb, k + 3)
            m01 = lane(mrow[0], k + 1)
            m02 = lane(mrow[0], k + 2)
            m03 = lane(mrow[0], k + 3)
            m12 = lane(mrow[1], k + 2)
            m13 = lane(mrow[1], k + 3)
            m23 = lane(mrow[2], k + 3)
            b = b_0 * (1.0 - m01 * a)
            c = c_0 * (1.0 - m02 * a) * (1.0 - m12 * b)
            d = d_0 * (1.0 - m03 * a) * (1.0 - m13 * b) * (1.0 - m23 * c)
            for u, ku in ((0, a), (1, b), (2, c), (3, d)):
                gtu = (iota_c > (k + u)).astype(f32)
                kb = kb * (1.0 - mrow[u] * gtu * ku)
            return kb

        kb = jax.lax.fori_loop(0, 32, step4, keep[0:1, pl.ds(b0, 128)])
        keep[0:1, pl.ds(b0, 128)] = kb
        mblock = mm[pl.ds(b0, 128), :]
        supp = jax.lax.dot_general(kb, mblock, (((1,), (0,)), ((), ())),
                                   preferred_element_type=f32)
        kv = keep[0:1, :]
        keep[0:1, :] = jnp.where((iota_j >= b0 + 128) & (supp > 0.0),
                                 0.0, kv)
        return 0

    jax.lax.fori_loop(0, _PREPAD // 128, nms_block, 0)

    kv = keep[0:1, :]
    ssv = ss[0:1, :]
    valid = iota_j < _PRE
    cond1 = (kv > 0.5) & (ssv > -5e8) & valid
    elig1 = jnp.where(cond1, 1.0, 0.0)
    elig2 = jnp.where(valid & (~cond1), 1.0, 0.0)

    def ut_tile(t, _):
        r0 = t * 8
        mm[pl.ds(r0, 8), :] = ((iota_s8 + r0) < iota_j).astype(f32)
        return 0

    jax.lax.fori_loop(0, _PREPAD // 8, ut_tile, 0)

    def vecmat(vec):
        acc = jnp.zeros((1, _PREPAD), f32)
        for t in range(8):
            acc = acc + jax.lax.dot_general(
                vec[0:1, t * 128:(t + 1) * 128],
                mm[pl.ds(t * 128, 128), :],
                (((1,), (0,)), ((), ())), preferred_element_type=f32)
        return acc

    rank1 = vecmat(elig1)
    rank2 = vecmat(elig2)
    total1 = jnp.sum(elig1, axis=1, keepdims=True)
    r_row = jnp.where(cond1, rank1,
                      jnp.where(elig2 > 0.0, rank2 + total1, 4096.0))
    sadj = jnp.where(cond1, ssv, jnp.float32(_NEG))

    xr1 = bsx1[0:1, :]
    yr1 = bsy1[0:1, :]
    xr2 = bsx2[0:1, :]
    yr2 = bsy2[0:1, :]
    for p in range(_POST):
        lmp = r_row == jnp.float32(p)
        x1v = jnp.sum(jnp.where(lmp, xr1, 0.0), axis=1, keepdims=True)
        y1v = jnp.sum(jnp.where(lmp, yr1, 0.0), axis=1, keepdims=True)
        x2v = jnp.sum(jnp.where(lmp, xr2, 0.0), axis=1, keepdims=True)
        y2v = jnp.sum(jnp.where(lmp, yr2, 0.0), axis=1, keepdims=True)
        sv = jnp.sum(jnp.where(lmp, sadj, 0.0), axis=1, keepdims=True)
        out_ref[p:p + 1, :] = jnp.where(
            iota_c == 0, x1v,
            jnp.where(iota_c == 1, y1v,
                      jnp.where(iota_c == 2, x2v,
                                jnp.where(iota_c == 3, y2v, sv))))


def _run(boxes, scores, interpret=False):
    f32 = jnp.float32
    pad = _NPAD - _N
    s2d = jnp.pad(scores, (0, pad)).reshape(_ROWS, 128)
    planes = [jnp.pad(boxes[:, k], (0, pad)).reshape(_ROWS, 128)
              for k in range(4)]
    out = pl.pallas_call(
        _nms_body,
        out_shape=jax.ShapeDtypeStruct((_POST, 128), f32),
        scratch_shapes=[
            pltpu.VMEM((_ROWS, 128), f32),
            pltpu.VMEM((1, _PREPAD), f32),
            pltpu.VMEM((1, _PREPAD), f32),
            pltpu.VMEM((1, _PREPAD), f32),
            pltpu.VMEM((1, _PREPAD), f32),
            pltpu.VMEM((1, _PREPAD), f32),
            pltpu.VMEM((_PREPAD, 1), f32),
            pltpu.VMEM((_PREPAD, 1), f32),
            pltpu.VMEM((_PREPAD, 1), f32),
            pltpu.VMEM((_PREPAD, 1), f32),
            pltpu.VMEM((_PREPAD, _PREPAD), f32),
            pltpu.VMEM((_PREPAD, 128), f32),
            pltpu.VMEM((1, _PREPAD), f32),
            pltpu.SMEM((1,), jnp.int32),
        ] + [pltpu.VMEM((_ROWS, 128), f32) for _ in range(_NB)],
        interpret=interpret,
    )(s2d, *planes)
    return out[:, :5]


@jax.jit
def _run_compiled(boxes, scores):
    return _run(boxes, scores)


def kernel(boxes, scores):
    return _run_compiled(boxes, scores)
